# baseline (speedup 1.0000x reference)
import functools

import jax
import jax.numpy as jnp
from jax import lax
from jax.experimental import pallas as pl
from jax.experimental.pallas import tpu as pltpu
from jax.experimental.pallas import tpu_sc as plsc

_TOP_K = (1, 5)
_THR = 0.0
_NA = 64


def _gather_targets(y_hat, y):
    B, V = y_hat.shape
    info = plsc.get_sparse_core_info()
    ncores, nsub, L = info.num_cores, info.num_subcores, info.num_lanes
    nw = ncores * nsub
    bw = B // nw
    flat = y_hat.reshape(B * V)
    mesh = plsc.VectorSubcoreMesh(core_axis_name="c", subcore_axis_name="s")

    @functools.partial(
        pl.kernel, mesh=mesh,
        out_type=jax.ShapeDtypeStruct((B,), jnp.float32),
        scratch_types=[
            pltpu.VMEM((bw,), jnp.int32),
            pltpu.VMEM((bw,), jnp.float32),
            pltpu.SemaphoreType.DMA,
        ],
    )
    def gat(flat_hbm, y_hbm, out_hbm, idx_v, vals_v, sem):
        wid = lax.axis_index("s") * ncores + lax.axis_index("c")
        base = wid * bw
        pltpu.sync_copy(y_hbm.at[pl.ds(base, bw)], idx_v)
        for k in range(bw // L):
            row0 = base + k * L
            off = (row0 + lax.iota(jnp.int32, L)) * V
            sl = pl.ds(k * L, L)
            idx_v[sl] = idx_v[sl] + off
        pltpu.async_copy(flat_hbm.at[idx_v], vals_v, sem).wait()
        pltpu.sync_copy(vals_v, out_hbm.at[pl.ds(base, bw)])

    return gat(flat, y).reshape(B, 1)


def _count_body(t_ref, x_ref, cg_ref):
    j = pl.program_id(1)

    @pl.when(j == 0)
    def _init():
        cg_ref[...] = jnp.zeros_like(cg_ref)

    v = x_ref[...]
    t = t_ref[...]
    cg_ref[...] += jnp.sum(jnp.where(v > t, 1, 0), axis=1, keepdims=True)


def _count_pass(y_hat, t):
    B, V = y_hat.shape
    R = min(512, B)
    nr = B // R
    C = min(8192, ((V + 127) // 128) * 128)
    nc_main = V // C
    vm = nc_main * C

    cg = pl.pallas_call(
        _count_body,
        grid=(nr, nc_main),
        in_specs=[
            pl.BlockSpec((R, 1), lambda i, j: (i, 0)),
            pl.BlockSpec((R, C), lambda i, j: (i, j)),
        ],
        out_specs=pl.BlockSpec((R, 1), lambda i, j: (i, 0)),
        out_shape=jax.ShapeDtypeStruct((B, 1), jnp.int32),
    )(t, y_hat)

    tail_w = V - vm
    if tail_w:
        ct = ((tail_w + 127) // 128) * 128
        tail = jnp.pad(y_hat[:, vm:], ((0, 0), (0, ct - tail_w)),
                       constant_values=-jnp.inf)
        cg_tail = pl.pallas_call(
            _count_body,
            grid=(nr, 1),
            in_specs=[
                pl.BlockSpec((R, 1), lambda i, j: (i, 0)),
                pl.BlockSpec((R, ct), lambda i, j: (i, j)),
            ],
            out_specs=pl.BlockSpec((R, 1), lambda i, j: (i, 0)),
            out_shape=jax.ShapeDtypeStruct((B, 1), jnp.int32),
        )(t, tail)
        cg = cg + cg_tail
    return cg


def _tie_body(rows_ref, t_ref, y_ref, x_ref, out_ref):
    v = x_ref[...]
    col = jax.lax.broadcasted_iota(jnp.int32, v.shape, 2)
    eql = (v == t_ref[...]) & (col < y_ref[...])
    cnt = jnp.sum(jnp.where(eql, 1, 0))
    out_ref[...] = cnt.reshape(1, 1, 1)


def _tie_pass(y_hat, rows, t_amb, y_amb):
    B, V = y_hat.shape
    x3 = y_hat.reshape(B, 1, V)
    grid_spec = pltpu.PrefetchScalarGridSpec(
        num_scalar_prefetch=1,
        grid=(_NA,),
        in_specs=[
            pl.BlockSpec((1, 1, 1), lambda g, r: (g, 0, 0)),
            pl.BlockSpec((1, 1, 1), lambda g, r: (g, 0, 0)),
            pl.BlockSpec((1, 1, V), lambda g, r: (r[g], 0, 0)),
        ],
        out_specs=pl.BlockSpec((1, 1, 1), lambda g, r: (g, 0, 0)),
    )
    out = pl.pallas_call(
        _tie_body,
        grid_spec=grid_spec,
        out_shape=jax.ShapeDtypeStruct((_NA, 1, 1), jnp.int32),
    )(rows, t_amb.reshape(_NA, 1, 1), y_amb.reshape(_NA, 1, 1), x3)
    return out.reshape(_NA)


def _final_body(num, scal_ref, cg_ref, t_ref, acc_ref):
    rank = cg_ref[...]
    rid = (jax.lax.broadcasted_iota(jnp.int32, rank.shape, 0) * 128
           + jax.lax.broadcasted_iota(jnp.int32, rank.shape, 1))
    for g in range(_NA):
        rank = rank + jnp.where(rid == scal_ref[g], scal_ref[_NA + g], 0)
    tpos = t_ref[...] > _THR
    c1 = jnp.sum(jnp.where((rank < _TOP_K[0]) & tpos, 1.0, 0.0))
    c5 = jnp.sum(jnp.where((rank < _TOP_K[1]) & tpos, 1.0, 0.0))
    lane2 = jax.lax.broadcasted_iota(jnp.int32, (1, 2), 1)
    acc_ref[...] = jnp.where(lane2 == 0, c1, c5) * (100.0 / num)


def _final_pass(cg_corrless, t, rows, corr):
    B = cg_corrless.shape[0]
    sub = B // 128
    scal = jnp.concatenate([rows, corr]).astype(jnp.int32)
    grid_spec = pltpu.PrefetchScalarGridSpec(
        num_scalar_prefetch=1,
        grid=(1,),
        in_specs=[
            pl.BlockSpec((sub, 128), lambda i, s: (0, 0)),
            pl.BlockSpec((sub, 128), lambda i, s: (0, 0)),
        ],
        out_specs=pl.BlockSpec((1, 2), lambda i, s: (0, 0)),
    )
    return pl.pallas_call(
        functools.partial(_final_body, B),
        grid_spec=grid_spec,
        out_shape=jax.ShapeDtypeStruct((1, 2), jnp.float32),
    )(scal, cg_corrless.reshape(sub, 128), t.reshape(sub, 128))


def kernel(y_hat, y):
    B, V = y_hat.shape
    y = y.astype(jnp.int32)
    t = _gather_targets(y_hat, y)
    cg = _count_pass(y_hat, t)

    amb = cg.reshape(B) <= max(_TOP_K) - 1
    rows = jnp.flatnonzero(amb, size=_NA, fill_value=0).astype(jnp.int32)
    slot_valid = jnp.arange(_NA) < jnp.sum(amb.astype(jnp.int32))
    t_amb = t.reshape(B)[rows]
    y_amb = y[rows]
    eq_lower = _tie_pass(y_hat, rows, t_amb, y_amb)
    corr = jnp.where(slot_valid, eq_lower, 0).astype(jnp.int32)

    acc = _final_pass(cg, t, rows, corr)
    return (acc[0, 0:1], acc[0, 1:2])


# speedup vs baseline: 1.8303x; 1.8303x over previous
import functools

import jax
import jax.numpy as jnp
from jax import lax
from jax.experimental import pallas as pl
from jax.experimental.pallas import tpu as pltpu
from jax.experimental.pallas import tpu_sc as plsc

_TOP_K = (1, 5)
_THR = 0.0
_NA = 64


def _gather_targets(y_hat, y):
    B, V = y_hat.shape
    info = plsc.get_sparse_core_info()
    ncores, nsub, L = info.num_cores, info.num_subcores, info.num_lanes
    nw = ncores * nsub
    bw = B // nw
    flat = y_hat.reshape(B * V)
    mesh = plsc.VectorSubcoreMesh(core_axis_name="c", subcore_axis_name="s")

    @functools.partial(
        pl.kernel, mesh=mesh,
        out_type=jax.ShapeDtypeStruct((B,), jnp.float32),
        scratch_types=[
            pltpu.VMEM((bw,), jnp.int32),
            pltpu.VMEM((bw,), jnp.float32),
            pltpu.SemaphoreType.DMA,
        ],
    )
    def gat(flat_hbm, y_hbm, out_hbm, idx_v, vals_v, sem):
        wid = lax.axis_index("s") * ncores + lax.axis_index("c")
        base = wid * bw
        pltpu.sync_copy(y_hbm.at[pl.ds(base, bw)], idx_v)
        for k in range(bw // L):
            row0 = base + k * L
            off = (row0 + lax.iota(jnp.int32, L)) * V
            sl = pl.ds(k * L, L)
            idx_v[sl] = idx_v[sl] + off
        pltpu.async_copy(flat_hbm.at[idx_v], vals_v, sem).wait()
        pltpu.sync_copy(vals_v, out_hbm.at[pl.ds(base, bw)])

    return gat(flat, y).reshape(B, 1)


def _count_body(t_ref, x_ref, cg_ref):
    j = pl.program_id(1)

    @pl.when(j == 0)
    def _init():
        cg_ref[...] = jnp.zeros_like(cg_ref)

    v = x_ref[...]
    t = t_ref[...]
    cg_ref[...] += jnp.sum(jnp.where(v > t, 1, 0), axis=1, keepdims=True)


def _count_pass(y_hat, t):
    B, V = y_hat.shape
    R = min(512, B)
    nr = B // R
    C = min(8192, ((V + 127) // 128) * 128)
    nc_main = V // C
    vm = nc_main * C

    cg = pl.pallas_call(
        _count_body,
        grid=(nr, nc_main),
        in_specs=[
            pl.BlockSpec((R, 1), lambda i, j: (i, 0)),
            pl.BlockSpec((R, C), lambda i, j: (i, j)),
        ],
        out_specs=pl.BlockSpec((R, 1), lambda i, j: (i, 0)),
        out_shape=jax.ShapeDtypeStruct((B, 1), jnp.int32),
    )(t, y_hat)

    tail_w = V - vm
    if tail_w:
        ct = ((tail_w + 127) // 128) * 128
        tail = jnp.pad(y_hat[:, vm:], ((0, 0), (0, ct - tail_w)),
                       constant_values=-jnp.inf)
        cg_tail = pl.pallas_call(
            _count_body,
            grid=(nr, 1),
            in_specs=[
                pl.BlockSpec((R, 1), lambda i, j: (i, 0)),
                pl.BlockSpec((R, ct), lambda i, j: (i, j)),
            ],
            out_specs=pl.BlockSpec((R, 1), lambda i, j: (i, 0)),
            out_shape=jax.ShapeDtypeStruct((B, 1), jnp.int32),
        )(t, tail)
        cg = cg + cg_tail
    return cg


def _tie_body(rows_ref, t_ref, y_ref, x_ref, out_ref):
    g = pl.program_id(0)
    v = x_ref[...]
    col = jax.lax.broadcasted_iota(jnp.int32, v.shape, 1)
    sub = jax.lax.broadcasted_iota(jnp.int32, v.shape, 0)
    inrow = sub == rows_ref[g] % 8
    eql = (v == t_ref[0, 0, 0]) & (col < y_ref[0, 0, 0]) & inrow
    cnt = jnp.sum(jnp.where(eql, 1, 0))
    out_ref[...] = cnt.reshape(1, 1, 1)


def _tie_pass(y_hat, rows, t_amb, y_amb):
    B, V = y_hat.shape
    grid_spec = pltpu.PrefetchScalarGridSpec(
        num_scalar_prefetch=1,
        grid=(_NA,),
        in_specs=[
            pl.BlockSpec((1, 1, 1), lambda g, r: (g, 0, 0)),
            pl.BlockSpec((1, 1, 1), lambda g, r: (g, 0, 0)),
            pl.BlockSpec((8, V), lambda g, r: (r[g] // 8, 0)),
        ],
        out_specs=pl.BlockSpec((1, 1, 1), lambda g, r: (g, 0, 0)),
    )
    out = pl.pallas_call(
        _tie_body,
        grid_spec=grid_spec,
        out_shape=jax.ShapeDtypeStruct((_NA, 1, 1), jnp.int32),
    )(rows, t_amb.reshape(_NA, 1, 1), y_amb.reshape(_NA, 1, 1), y_hat)
    return out.reshape(_NA)


def _final_body(num, scal_ref, cg_ref, t_ref, acc_ref):
    rank = cg_ref[...]
    rid = (jax.lax.broadcasted_iota(jnp.int32, rank.shape, 0) * 128
           + jax.lax.broadcasted_iota(jnp.int32, rank.shape, 1))
    for g in range(_NA):
        rank = rank + jnp.where(rid == scal_ref[g], scal_ref[_NA + g], 0)
    tpos = t_ref[...] > _THR
    c1 = jnp.sum(jnp.where((rank < _TOP_K[0]) & tpos, 1.0, 0.0))
    c5 = jnp.sum(jnp.where((rank < _TOP_K[1]) & tpos, 1.0, 0.0))
    lane2 = jax.lax.broadcasted_iota(jnp.int32, (1, 2), 1)
    acc_ref[...] = jnp.where(lane2 == 0, c1, c5) * (100.0 / num)


def _final_pass(cg_corrless, t, rows, corr):
    B = cg_corrless.shape[0]
    sub = B // 128
    scal = jnp.concatenate([rows, corr]).astype(jnp.int32)
    grid_spec = pltpu.PrefetchScalarGridSpec(
        num_scalar_prefetch=1,
        grid=(1,),
        in_specs=[
            pl.BlockSpec((sub, 128), lambda i, s: (0, 0)),
            pl.BlockSpec((sub, 128), lambda i, s: (0, 0)),
        ],
        out_specs=pl.BlockSpec((1, 2), lambda i, s: (0, 0)),
    )
    return pl.pallas_call(
        functools.partial(_final_body, B),
        grid_spec=grid_spec,
        out_shape=jax.ShapeDtypeStruct((1, 2), jnp.float32),
    )(scal, cg_corrless.reshape(sub, 128), t.reshape(sub, 128))


def kernel(y_hat, y):
    B, V = y_hat.shape
    y = y.astype(jnp.int32)
    t = _gather_targets(y_hat, y)
    cg = _count_pass(y_hat, t)

    amb = cg.reshape(B) <= max(_TOP_K) - 1
    rows = jnp.flatnonzero(amb, size=_NA, fill_value=0).astype(jnp.int32)
    slot_valid = jnp.arange(_NA) < jnp.sum(amb.astype(jnp.int32))
    t_amb = t.reshape(B)[rows]
    y_amb = y[rows]
    eq_lower = _tie_pass(y_hat, rows, t_amb, y_amb)
    corr = jnp.where(slot_valid, eq_lower, 0).astype(jnp.int32)

    acc = _final_pass(cg, t, rows, corr)
    return (acc[0, 0:1], acc[0, 1:2])


# speedup vs baseline: 1.8317x; 1.0007x over previous
import functools

import jax
import jax.numpy as jnp
from jax import lax
from jax.experimental import pallas as pl
from jax.experimental.pallas import tpu as pltpu
from jax.experimental.pallas import tpu_sc as plsc

_TOP_K = (1, 5)
_THR = 0.0
_NA = 64


def _gather_targets(y_hat, y):
    B, V = y_hat.shape
    info = plsc.get_sparse_core_info()
    ncores, nsub, L = info.num_cores, info.num_subcores, info.num_lanes
    nw = ncores * nsub
    bw = B // nw
    flat = y_hat.reshape(B * V)
    mesh = plsc.VectorSubcoreMesh(core_axis_name="c", subcore_axis_name="s")

    @functools.partial(
        pl.kernel, mesh=mesh,
        out_type=jax.ShapeDtypeStruct((B,), jnp.float32),
        scratch_types=[
            pltpu.VMEM((bw,), jnp.int32),
            pltpu.VMEM((bw,), jnp.float32),
            pltpu.SemaphoreType.DMA,
        ],
    )
    def gat(flat_hbm, y_hbm, out_hbm, idx_v, vals_v, sem):
        wid = lax.axis_index("s") * ncores + lax.axis_index("c")
        base = wid * bw
        pltpu.sync_copy(y_hbm.at[pl.ds(base, bw)], idx_v)
        for k in range(bw // L):
            row0 = base + k * L
            off = (row0 + lax.iota(jnp.int32, L)) * V
            sl = pl.ds(k * L, L)
            idx_v[sl] = idx_v[sl] + off
        pltpu.async_copy(flat_hbm.at[idx_v], vals_v, sem).wait()
        pltpu.sync_copy(vals_v, out_hbm.at[pl.ds(base, bw)])

    return gat(flat, y).reshape(B, 1)


def _count_body(t_ref, x_ref, cg_ref):
    j = pl.program_id(1)

    @pl.when(j == 0)
    def _init():
        cg_ref[...] = jnp.zeros_like(cg_ref)

    v = x_ref[...]
    t = t_ref[...]
    cg_ref[...] += jnp.sum(jnp.where(v > t, 1, 0), axis=1, keepdims=True)


def _count_pass(y_hat, t):
    B, V = y_hat.shape
    R = min(512, B)
    nr = B // R
    C = min(8192, ((V + 127) // 128) * 128)
    nc_main = V // C
    vm = nc_main * C

    cg = pl.pallas_call(
        _count_body,
        grid=(nr, nc_main),
        compiler_params=pltpu.CompilerParams(
            dimension_semantics=("parallel", "arbitrary")),
        in_specs=[
            pl.BlockSpec((R, 1), lambda i, j: (i, 0)),
            pl.BlockSpec((R, C), lambda i, j: (i, j)),
        ],
        out_specs=pl.BlockSpec((R, 1), lambda i, j: (i, 0)),
        out_shape=jax.ShapeDtypeStruct((B, 1), jnp.int32),
    )(t, y_hat)

    tail_w = V - vm
    if tail_w:
        ct = ((tail_w + 127) // 128) * 128
        tail = jnp.pad(y_hat[:, vm:], ((0, 0), (0, ct - tail_w)),
                       constant_values=-jnp.inf)
        cg_tail = pl.pallas_call(
            _count_body,
            grid=(nr, 1),
            in_specs=[
                pl.BlockSpec((R, 1), lambda i, j: (i, 0)),
                pl.BlockSpec((R, ct), lambda i, j: (i, j)),
            ],
            out_specs=pl.BlockSpec((R, 1), lambda i, j: (i, 0)),
            out_shape=jax.ShapeDtypeStruct((B, 1), jnp.int32),
        )(t, tail)
        cg = cg + cg_tail
    return cg


def _tie_body(rows_ref, t_ref, y_ref, x_ref, out_ref):
    g = pl.program_id(0)
    v = x_ref[...]
    col = jax.lax.broadcasted_iota(jnp.int32, v.shape, 1)
    sub = jax.lax.broadcasted_iota(jnp.int32, v.shape, 0)
    inrow = sub == rows_ref[g] % 8
    eql = (v == t_ref[0, 0, 0]) & (col < y_ref[0, 0, 0]) & inrow
    cnt = jnp.sum(jnp.where(eql, 1, 0))
    out_ref[...] = cnt.reshape(1, 1, 1)


def _tie_pass(y_hat, rows, t_amb, y_amb):
    B, V = y_hat.shape
    grid_spec = pltpu.PrefetchScalarGridSpec(
        num_scalar_prefetch=1,
        grid=(_NA,),
        in_specs=[
            pl.BlockSpec((1, 1, 1), lambda g, r: (g, 0, 0)),
            pl.BlockSpec((1, 1, 1), lambda g, r: (g, 0, 0)),
            pl.BlockSpec((8, V), lambda g, r: (r[g] // 8, 0)),
        ],
        out_specs=pl.BlockSpec((1, 1, 1), lambda g, r: (g, 0, 0)),
    )
    out = pl.pallas_call(
        _tie_body,
        grid_spec=grid_spec,
        out_shape=jax.ShapeDtypeStruct((_NA, 1, 1), jnp.int32),
    )(rows, t_amb.reshape(_NA, 1, 1), y_amb.reshape(_NA, 1, 1), y_hat)
    return out.reshape(_NA)


def _final_body(num, scal_ref, cg_ref, t_ref, acc_ref):
    rank = cg_ref[...]
    rid = (jax.lax.broadcasted_iota(jnp.int32, rank.shape, 0) * 128
           + jax.lax.broadcasted_iota(jnp.int32, rank.shape, 1))
    for g in range(_NA):
        rank = rank + jnp.where(rid == scal_ref[g], scal_ref[_NA + g], 0)
    tpos = t_ref[...] > _THR
    c1 = jnp.sum(jnp.where((rank < _TOP_K[0]) & tpos, 1.0, 0.0))
    c5 = jnp.sum(jnp.where((rank < _TOP_K[1]) & tpos, 1.0, 0.0))
    lane2 = jax.lax.broadcasted_iota(jnp.int32, (1, 2), 1)
    acc_ref[...] = jnp.where(lane2 == 0, c1, c5) * (100.0 / num)


def _final_pass(cg_corrless, t, rows, corr):
    B = cg_corrless.shape[0]
    sub = B // 128
    scal = jnp.concatenate([rows, corr]).astype(jnp.int32)
    grid_spec = pltpu.PrefetchScalarGridSpec(
        num_scalar_prefetch=1,
        grid=(1,),
        in_specs=[
            pl.BlockSpec((sub, 128), lambda i, s: (0, 0)),
            pl.BlockSpec((sub, 128), lambda i, s: (0, 0)),
        ],
        out_specs=pl.BlockSpec((1, 2), lambda i, s: (0, 0)),
    )
    return pl.pallas_call(
        functools.partial(_final_body, B),
        grid_spec=grid_spec,
        out_shape=jax.ShapeDtypeStruct((1, 2), jnp.float32),
    )(scal, cg_corrless.reshape(sub, 128), t.reshape(sub, 128))


def kernel(y_hat, y):
    B, V = y_hat.shape
    y = y.astype(jnp.int32)
    t = _gather_targets(y_hat, y)
    cg = _count_pass(y_hat, t)

    amb = cg.reshape(B) <= max(_TOP_K) - 1
    rows = jnp.flatnonzero(amb, size=_NA, fill_value=0).astype(jnp.int32)
    slot_valid = jnp.arange(_NA) < jnp.sum(amb.astype(jnp.int32))
    t_amb = t.reshape(B)[rows]
    y_amb = y[rows]
    eq_lower = _tie_pass(y_hat, rows, t_amb, y_amb)
    corr = jnp.where(slot_valid, eq_lower, 0).astype(jnp.int32)

    acc = _final_pass(cg, t, rows, corr)
    return (acc[0, 0:1], acc[0, 1:2])
